# Initial kernel scaffold; baseline (speedup 1.0000x reference)
import jax
import jax.numpy as jnp
from jax.experimental import pallas as pl


def kernel(bilateral_grid, guidemap):
    raise NotImplementedError("write your pallas kernel here")


# speedup vs baseline: 55.2573x; 55.2573x over previous
import functools

import jax
import jax.numpy as jnp
from jax import lax
from jax.experimental import pallas as pl
from jax.experimental.pallas import tpu as pltpu
from jax.experimental.pallas import tpu_sc as plsc

B = 4
H = 512
W = 512
GH = 16
GW = 16
GD = 8
GC = 12
GRID_WORDS = GC * GD * GH * GW

NW = 32
ROWS_PER_W = H * B // NW
BLK = 8
N_BLK = ROWS_PER_W // BLK
GROUPS = W // 16


def _sc_body(grid_hbm, guide_hbm, out_hbm, grid_v, guide_v, out_v):
    c = lax.axis_index("c")
    s = lax.axis_index("s")
    wid = s * 2 + c
    b = wid // (NW // B)
    q = wid % (NW // B)

    pltpu.sync_copy(grid_hbm.at[b], grid_v)

    lane = lax.iota(jnp.int32, 16).astype(jnp.float32)
    wj1_odd = (lane + 0.5) * (1.0 / 32.0)
    wj1_even = wj1_odd + 0.5
    wj0_odd = 1.0 - wj1_odd
    wj0_even = 1.0 - wj1_even

    def blk_body(blk, carry):
        i0 = q * ROWS_PER_W + blk * BLK
        pltpu.sync_copy(guide_hbm.at[b, pl.ds(i0, BLK)], guide_v)

        def row_body(r, carry2):
            i = i0 + r
            gi0 = (i + 16) // 32 - 1
            fi = gi0.astype(jnp.float32)
            wi1 = (i.astype(jnp.float32) + 0.5) * (1.0 / 32.0) - fi - 0.5
            wi0 = 1.0 - wi1
            row0 = jnp.maximum(gi0, 0) * GW
            row1 = jnp.minimum(gi0 + 1, GH - 1) * GW

            def do_group(g, gj0, wj0v, wj1v):
                col0 = jnp.maximum(gj0, 0)
                col1 = jnp.minimum(gj0 + 1, GW - 1)
                c00 = row0 + col0
                c01 = row0 + col1
                c10 = row1 + col0
                c11 = row1 + col1

                gv = guide_v[r, pl.ds(g * 16, 16)]
                kf = gv * float(GD)
                k0i = (kf + 0.5).astype(jnp.int32)
                wk1 = kf - k0i.astype(jnp.float32) + 0.5
                wk0 = 1.0 - wk1
                kv0 = jnp.clip(k0i - 1, 0, GD - 1) * (GH * GW)
                kv1 = jnp.clip(k0i, 0, GD - 1) * (GH * GW)

                w00 = wi0 * wj0v
                w01 = wi0 * wj1v
                w10 = wi1 * wj0v
                w11 = wi1 * wj1v
                wv = (w00 * wk0, w00 * wk1, w01 * wk0, w01 * wk1,
                      w10 * wk0, w10 * wk1, w11 * wk0, w11 * wk1)
                iv = (kv0 + c00, kv1 + c00, kv0 + c01, kv1 + c01,
                      kv0 + c10, kv1 + c10, kv0 + c11, kv1 + c11)

                for ch in range(GC):
                    off = ch * (GD * GH * GW)
                    acc = wv[0] * plsc.load_gather(grid_v, [iv[0] + off])
                    for t in range(1, 8):
                        acc = acc + wv[t] * plsc.load_gather(
                            grid_v, [iv[t] + off])
                    out_v[ch, r, pl.ds(g * 16, 16)] = acc

            def grp_body(m, carry3):
                do_group(2 * m, m - 1, wj0_even, wj1_even)
                do_group(2 * m + 1, m, wj0_odd, wj1_odd)
                return carry3

            lax.fori_loop(0, GROUPS // 2, grp_body, 0)
            return carry2

        lax.fori_loop(0, BLK, row_body, 0)
        for ch in range(GC):
            pltpu.sync_copy(out_v.at[ch], out_hbm.at[b, ch, pl.ds(i0, BLK)])
        return carry

    lax.fori_loop(0, N_BLK, blk_body, 0)


def kernel(bilateral_grid, guidemap):
    grid_flat = bilateral_grid.reshape(B, GRID_WORDS)
    guide = guidemap.reshape(B, H, W)
    mesh = plsc.VectorSubcoreMesh(core_axis_name="c", subcore_axis_name="s")
    f = functools.partial(
        pl.kernel,
        mesh=mesh,
        compiler_params=pltpu.CompilerParams(needs_layout_passes=False),
        out_type=jax.ShapeDtypeStruct((B, GC, H, W), jnp.float32),
        scratch_types=[
            pltpu.VMEM((GRID_WORDS,), jnp.float32),
            pltpu.VMEM((BLK, W), jnp.float32),
            pltpu.VMEM((GC, BLK, W), jnp.float32),
        ],
    )(_sc_body)
    return f(grid_flat, guide)
